# baseline (speedup 1.0000x reference)
import jax
import jax.numpy as jnp
from jax.experimental import pallas as pl
from jax.experimental.pallas import tpu as pltpu

D_MODEL = 2048
NUM_EXPERTS = 64
N_TOKENS = 8192
CONFIDENCE_THRESHOLD = 0.5
TOKEN_TILE = 512
PAIR_ROWS = TOKEN_TILE // 2
LANES = 2 * NUM_EXPERTS


def _lane_group_iota():
    return jax.lax.broadcasted_iota(jnp.int32, (1, LANES), 1) & (NUM_EXPERTS - 1)


def _sort_desc_groups(v):
    idx = _lane_group_iota()
    k = 2
    while k <= NUM_EXPERTS:
        d = (idx & k) != 0
        j = k // 2
        while j >= 1:
            m = (idx & j) != 0
            pv = jnp.where(m, pltpu.roll(v, j, 1), pltpu.roll(v, LANES - j, 1))
            v = jnp.where(m == d, jnp.maximum(v, pv), jnp.minimum(v, pv))
            j //= 2
        k *= 2
    return v


def _cumsum_groups(v):
    idx = _lane_group_iota()
    s = 1
    while s < NUM_EXPERTS:
        v = v + jnp.where(idx >= s, pltpu.roll(v, s, 1), 0.0)
        s *= 2
    return v


def _butterfly(v, combine):
    idx = _lane_group_iota()
    s = 1
    while s < NUM_EXPERTS:
        pv = jnp.where((idx & s) != 0,
                       pltpu.roll(v, s, 1), pltpu.roll(v, LANES - s, 1))
        v = combine(v, pv)
        s *= 2
    return v


def _router_kernel(x_ref, w_ref, b_ref, rw_ref, probs_ref, cnt_ref):
    logits = jnp.dot(x_ref[...], w_ref[...],
                     preferred_element_type=jnp.float32,
                     precision=jax.lax.Precision.DEFAULT)
    logits = logits + b_ref[...]

    gi = jax.lax.broadcasted_iota(jnp.int32, (LANES, LANES), 0)
    gj = jax.lax.broadcasted_iota(jnp.int32, (LANES, LANES), 1)
    bd = ((gi // NUM_EXPERTS) == (gj // NUM_EXPERTS)).astype(jnp.float32)

    def gsum(a):
        return jnp.dot(a, bd, preferred_element_type=jnp.float32,
                       precision=jax.lax.Precision.HIGHEST)

    mx = _butterfly(logits, jnp.maximum)
    ex = jnp.exp(logits - mx)
    p = ex / gsum(ex)

    sp = _sort_desc_groups(p)
    shifted = _cumsum_groups(sp) - sp
    act_s = shifted < CONFIDENCE_THRESHOLD
    act_p_s = jnp.where(act_s, sp, 0.0)
    mass = gsum(act_p_s)
    theta = _butterfly(jnp.where(act_s, sp, jnp.inf), jnp.minimum)
    r = gsum((act_s & (sp == theta)).astype(jnp.float32))

    eqf = (p == theta).astype(jnp.float32)
    rank_excl = _cumsum_groups(eqf) - eqf
    active = (p > theta) | ((p == theta) & (rank_excl < r))

    active_probs = jnp.where(active, p, 0.0)
    rw_ref[...] = active_probs / (mass + 1e-6)
    probs_ref[...] = p
    cnt_ref[...] = gsum(active.astype(jnp.float32)).astype(jnp.int32)


def kernel(x, W, b):
    n_tiles = N_TOKENS // TOKEN_TILE
    xp = x.reshape(N_TOKENS // 2, 2 * D_MODEL)
    w2 = jnp.zeros((2 * D_MODEL, LANES), dtype=W.dtype)
    w2 = w2.at[:D_MODEL, :NUM_EXPERTS].set(W)
    w2 = w2.at[D_MODEL:, NUM_EXPERTS:].set(W)
    b2 = jnp.concatenate([b, b]).reshape(1, LANES)
    rw, probs, cnt = pl.pallas_call(
        _router_kernel,
        grid=(n_tiles,),
        in_specs=[
            pl.BlockSpec((PAIR_ROWS, 2 * D_MODEL), lambda i: (i, 0)),
            pl.BlockSpec((2 * D_MODEL, LANES), lambda i: (0, 0)),
            pl.BlockSpec((1, LANES), lambda i: (0, 0)),
        ],
        out_specs=[
            pl.BlockSpec((PAIR_ROWS, LANES), lambda i: (i, 0)),
            pl.BlockSpec((PAIR_ROWS, LANES), lambda i: (i, 0)),
            pl.BlockSpec((PAIR_ROWS, LANES), lambda i: (i, 0)),
        ],
        out_shape=[
            jax.ShapeDtypeStruct((N_TOKENS // 2, LANES), jnp.float32),
            jax.ShapeDtypeStruct((N_TOKENS // 2, LANES), jnp.float32),
            jax.ShapeDtypeStruct((N_TOKENS // 2, LANES), jnp.int32),
        ],
    )(xp, w2, b2)
    rw = rw.reshape(N_TOKENS, NUM_EXPERTS)
    probs = probs.reshape(N_TOKENS, NUM_EXPERTS)
    cnt = cnt.reshape(N_TOKENS, NUM_EXPERTS)[:, 0]
    return rw, probs, cnt


# speedup vs baseline: 1.0410x; 1.0410x over previous
import jax
import jax.numpy as jnp
from jax.experimental import pallas as pl
from jax.experimental.pallas import tpu as pltpu

D_MODEL = 2048
NUM_EXPERTS = 64
N_TOKENS = 8192
CONFIDENCE_THRESHOLD = 0.5
TOKEN_TILE = 2048
PAIR_ROWS = TOKEN_TILE // 2
LANES = 2 * NUM_EXPERTS


def _lane_group_iota():
    return jax.lax.broadcasted_iota(jnp.int32, (1, LANES), 1) & (NUM_EXPERTS - 1)


def _sort_desc_groups(v):
    idx = _lane_group_iota()
    k = 2
    while k <= NUM_EXPERTS:
        d = (idx & k) != 0
        j = k // 2
        while j >= 1:
            m = (idx & j) != 0
            pv = jnp.where(m, pltpu.roll(v, j, 1), pltpu.roll(v, LANES - j, 1))
            v = jnp.where(m == d, jnp.maximum(v, pv), jnp.minimum(v, pv))
            j //= 2
        k *= 2
    return v


def _cumsum_groups(v):
    idx = _lane_group_iota()
    s = 1
    while s < NUM_EXPERTS:
        v = v + jnp.where(idx >= s, pltpu.roll(v, s, 1), 0.0)
        s *= 2
    return v


def _butterfly(v, combine):
    idx = _lane_group_iota()
    s = 1
    while s < NUM_EXPERTS:
        pv = jnp.where((idx & s) != 0,
                       pltpu.roll(v, s, 1), pltpu.roll(v, LANES - s, 1))
        v = combine(v, pv)
        s *= 2
    return v


def _router_kernel(x_ref, w_ref, b_ref, rw_ref, probs_ref, cnt_ref):
    logits = jnp.dot(x_ref[...], w_ref[...],
                     preferred_element_type=jnp.float32,
                     precision=jax.lax.Precision.DEFAULT)
    logits = logits + b_ref[...]

    gi = jax.lax.broadcasted_iota(jnp.int32, (LANES, LANES), 0)
    gj = jax.lax.broadcasted_iota(jnp.int32, (LANES, LANES), 1)
    bd = ((gi // NUM_EXPERTS) == (gj // NUM_EXPERTS)).astype(jnp.float32)

    def gsum(a):
        return jnp.dot(a, bd, preferred_element_type=jnp.float32,
                       precision=jax.lax.Precision.HIGHEST)

    mx = _butterfly(logits, jnp.maximum)
    ex = jnp.exp(logits - mx)
    p = ex / gsum(ex)

    sp = _sort_desc_groups(p)
    shifted = _cumsum_groups(sp) - sp
    act_s = shifted < CONFIDENCE_THRESHOLD
    act_p_s = jnp.where(act_s, sp, 0.0)
    mass = gsum(act_p_s)
    theta = _butterfly(jnp.where(act_s, sp, jnp.inf), jnp.minimum)
    r = gsum((act_s & (sp == theta)).astype(jnp.float32))

    eqf = (p == theta).astype(jnp.float32)
    rank_excl = _cumsum_groups(eqf) - eqf
    active = (p > theta) | ((p == theta) & (rank_excl < r))

    active_probs = jnp.where(active, p, 0.0)
    rw_ref[...] = active_probs / (mass + 1e-6)
    probs_ref[...] = p
    cnt_ref[...] = gsum(active.astype(jnp.float32)).astype(jnp.int32)


def kernel(x, W, b):
    n_tiles = N_TOKENS // TOKEN_TILE
    xp = x.reshape(N_TOKENS // 2, 2 * D_MODEL)
    w2 = jnp.zeros((2 * D_MODEL, LANES), dtype=W.dtype)
    w2 = w2.at[:D_MODEL, :NUM_EXPERTS].set(W)
    w2 = w2.at[D_MODEL:, NUM_EXPERTS:].set(W)
    b2 = jnp.concatenate([b, b]).reshape(1, LANES)
    rw, probs, cnt = pl.pallas_call(
        _router_kernel,
        grid=(n_tiles,),
        in_specs=[
            pl.BlockSpec((PAIR_ROWS, 2 * D_MODEL), lambda i: (i, 0)),
            pl.BlockSpec((2 * D_MODEL, LANES), lambda i: (0, 0)),
            pl.BlockSpec((1, LANES), lambda i: (0, 0)),
        ],
        out_specs=[
            pl.BlockSpec((PAIR_ROWS, LANES), lambda i: (i, 0)),
            pl.BlockSpec((PAIR_ROWS, LANES), lambda i: (i, 0)),
            pl.BlockSpec((PAIR_ROWS, LANES), lambda i: (i, 0)),
        ],
        out_shape=[
            jax.ShapeDtypeStruct((N_TOKENS // 2, LANES), jnp.float32),
            jax.ShapeDtypeStruct((N_TOKENS // 2, LANES), jnp.float32),
            jax.ShapeDtypeStruct((N_TOKENS // 2, LANES), jnp.int32),
        ],
    )(xp, w2, b2)
    rw = rw.reshape(N_TOKENS, NUM_EXPERTS)
    probs = probs.reshape(N_TOKENS, NUM_EXPERTS)
    cnt = cnt.reshape(N_TOKENS, NUM_EXPERTS)[:, 0]
    return rw, probs, cnt
